# baseline (speedup 1.0000x reference)
import functools

import jax
import jax.numpy as jnp
from jax.experimental import pallas as pl
from jax.experimental.pallas import tpu as pltpu

_EPS = 1e-5
_SLOPE = 0.2
_VMEM_LIMIT = 64 * 1024 * 1024


def _lrelu(v):
    return jnp.where(v >= 0.0, v, _SLOPE * v)


def _taps_stats(e, o, wa_ref, wb_ref, out_ref, s1_ref, s2_ref, ep_ref, op_ref,
                *, S, B):
    c2 = e.shape[3]
    c = c2 // 2
    zero = jnp.zeros((B, S + 1, S + 2, c2), e.dtype)
    ep_ref[...] = zero
    op_ref[...] = zero
    ep_ref[:, 0:S, 1 : S + 1, :] = e
    op_ref[:, 1 : S + 1, 1 : S + 1, :] = o
    refs = (op_ref, ep_ref, op_ref, ep_ref)
    offs = (0, 0, 1, 1)
    acc = None
    for i in range(4):
        r = refs[i]
        a_sl = r[:, offs[i] : offs[i] + S, 1 : S + 1, :]
        d = jnp.dot(a_sl.reshape(B * S * S, c2), wa_ref[i],
                    preferred_element_type=jnp.float32)
        acc = d if acc is None else acc + d
        b_sl = jnp.concatenate(
            [r[:, offs[i] : offs[i] + S, 0:S, c:],
             r[:, offs[i] : offs[i] + S, 2 : S + 2, :c]], axis=3)
        acc = acc + jnp.dot(b_sl.reshape(B * S * S, c2), wb_ref[i],
                            preferred_element_type=jnp.float32)
    out_ref[...] = acc.astype(out_ref.dtype)
    s1 = jnp.sum(acc, axis=0, keepdims=True)
    s2 = jnp.sum(acc * acc, axis=0, keepdims=True)
    row = jax.lax.broadcasted_iota(jnp.int32, s1_ref.shape, 0)
    s1_ref[...] = jnp.where(row == 0, jnp.broadcast_to(s1, s1_ref.shape), 0.0)
    s2_ref[...] = jnp.where(row == 0, jnp.broadcast_to(s2, s2_ref.shape), 0.0)


def _l1l2_kernel(p_ref, w1_ref, wa_ref, wb_ref, out_ref, s1_ref, s2_ref,
                 ep_ref, op_ref, *, S, B):
    pv = p_ref[...].reshape(B, 16, 2, 16, 32)
    acts = []
    for ph in (0, 1):
        pp = pv[:, :, ph].reshape(B * 16 * 16, 32)
        a = jnp.dot(pp, w1_ref[...], preferred_element_type=jnp.float32)
        acts.append(_lrelu(a).astype(jnp.bfloat16).reshape(B, 16, 16, 128))
    _taps_stats(acts[0], acts[1], wa_ref, wb_ref, out_ref, s1_ref, s2_ref,
                ep_ref, op_ref, S=S, B=B)


def _fused_conv_kernel(y_ref, sc_ref, sh_ref, wa_ref, wb_ref,
                       out_ref, s1_ref, s2_ref, ep_ref, op_ref,
                       *, S, B, has_bn):
    c2 = y_ref.shape[3]
    yv = y_ref[...].reshape(B, S, 2, S, c2)
    e = yv[:, :, 0]
    o = yv[:, :, 1]
    if has_bn:
        e = _lrelu(e.astype(jnp.float32) * sc_ref[...] + sh_ref[...])
        o = _lrelu(o.astype(jnp.float32) * sc_ref[...] + sh_ref[...])
        e = e.astype(jnp.bfloat16)
        o = o.astype(jnp.bfloat16)
    _taps_stats(e, o, wa_ref, wb_ref, out_ref, s1_ref, s2_ref,
                ep_ref, op_ref, S=S, B=B)


def _head_kernel(y_ref, sc_ref, sh_ref, w5_ref, o_ref, *, B):
    z = _lrelu(y_ref[...].astype(jnp.float32) * sc_ref[...] + sh_ref[...])
    zw = z * w5_ref[...]
    t = zw.reshape(B, 16, 512)
    s = jnp.sum(t, axis=2)
    logit = jnp.sum(s, axis=1, keepdims=True)
    prob = 1.0 / (1.0 + jnp.exp(-logit))
    o_ref[...] = jnp.broadcast_to(prob, o_ref.shape)


def _fold_stats(s1p, s2p, gamma, beta, m):
    s1 = jnp.sum(s1p, axis=0)
    s2 = jnp.sum(s2p, axis=0)
    mean = s1 / m
    var = jnp.maximum(s2 / m - mean * mean, 0.0)
    sc = gamma.astype(jnp.float32) * jax.lax.rsqrt(var + _EPS)
    sh = beta.astype(jnp.float32) - mean * sc
    return sc.reshape(1, -1), sh.reshape(1, -1)


def _l1l2_conv(x_nhwc, w1, w2, *, B=8):
    n = x_nhwc.shape[0]
    B = min(B, n)
    g = n // B
    oc = w2.shape[3]
    m = n * 256
    xp = jnp.pad(x_nhwc.astype(jnp.bfloat16), ((0, 0), (1, 1), (1, 1), (0, 0)))
    cols = [xp[:, i : i + 64 : 2, j : j + 64 : 2, 0] for i in range(4) for j in range(4)]
    patches = jnp.stack(cols, axis=-1)
    pv = patches.reshape(n, 32, 16, 32)

    wf = w1.reshape(16, 64)
    w1bd = jnp.zeros((32, 128), jnp.float32)
    w1bd = w1bd.at[0:16, 0:64].set(wf)
    w1bd = w1bd.at[16:32, 64:128].set(wf)
    w1bd = w1bd.astype(jnp.bfloat16)

    wr = w2.astype(jnp.bfloat16)
    wa = jnp.concatenate([wr[:, 1], wr[:, 2]], axis=1)
    wb = jnp.concatenate([wr[:, 0], wr[:, 3]], axis=1)

    y, s1, s2 = pl.pallas_call(
        functools.partial(_l1l2_kernel, S=16, B=B),
        out_shape=(jax.ShapeDtypeStruct((m, oc), jnp.bfloat16),
                   jax.ShapeDtypeStruct((8 * g, oc), jnp.float32),
                   jax.ShapeDtypeStruct((8 * g, oc), jnp.float32)),
        grid=(g,),
        in_specs=[pl.BlockSpec((B, 32, 16, 32), lambda i: (i, 0, 0, 0)),
                  pl.BlockSpec((32, 128), lambda i: (0, 0)),
                  pl.BlockSpec((4, 128, oc), lambda i: (0, 0, 0)),
                  pl.BlockSpec((4, 128, oc), lambda i: (0, 0, 0))],
        out_specs=(pl.BlockSpec((B * 256, oc), lambda i: (i, 0)),
                   pl.BlockSpec((8, oc), lambda i: (i, 0)),
                   pl.BlockSpec((8, oc), lambda i: (i, 0))),
        scratch_shapes=[pltpu.VMEM((B, 17, 18, 128), jnp.bfloat16),
                        pltpu.VMEM((B, 17, 18, 128), jnp.bfloat16)],
        compiler_params=pltpu.CompilerParams(
            dimension_semantics=("parallel",), vmem_limit_bytes=_VMEM_LIMIT),
    )(pv, w1bd, wa, wb)
    return y.reshape(n, 16, 16, oc), s1, s2


def _fused_conv(y_prev, sc, sh, w, *, S, B, has_bn=True):
    n, _, _, c = y_prev.shape
    oc = w.shape[3]
    B = min(B, n)
    g = n // B
    m = n * S * S
    c2 = 2 * c
    yv = y_prev.reshape(n, 2 * S, S, c2)
    wr = w.astype(jnp.bfloat16)
    wa = jnp.concatenate([wr[:, 1], wr[:, 2]], axis=1)
    wb = jnp.concatenate([wr[:, 0], wr[:, 3]], axis=1)
    scd = jnp.tile(sc, (1, 2))
    shd = jnp.tile(sh, (1, 2))
    body = functools.partial(_fused_conv_kernel, S=S, B=B, has_bn=has_bn)
    y, s1, s2 = pl.pallas_call(
        body,
        out_shape=(jax.ShapeDtypeStruct((m, oc), jnp.bfloat16),
                   jax.ShapeDtypeStruct((8 * g, oc), jnp.float32),
                   jax.ShapeDtypeStruct((8 * g, oc), jnp.float32)),
        grid=(g,),
        in_specs=[pl.BlockSpec((B, 2 * S, S, c2), lambda i: (i, 0, 0, 0)),
                  pl.BlockSpec((1, c2), lambda i: (0, 0)),
                  pl.BlockSpec((1, c2), lambda i: (0, 0)),
                  pl.BlockSpec((4, c2, oc), lambda i: (0, 0, 0)),
                  pl.BlockSpec((4, c2, oc), lambda i: (0, 0, 0))],
        out_specs=(pl.BlockSpec((B * S * S, oc), lambda i: (i, 0)),
                   pl.BlockSpec((8, oc), lambda i: (i, 0)),
                   pl.BlockSpec((8, oc), lambda i: (i, 0))),
        scratch_shapes=[pltpu.VMEM((B, S + 1, S + 2, c2), jnp.bfloat16),
                        pltpu.VMEM((B, S + 1, S + 2, c2), jnp.bfloat16)],
        compiler_params=pltpu.CompilerParams(
            dimension_semantics=("parallel",), vmem_limit_bytes=_VMEM_LIMIT),
    )(yv, scd, shd, wa, wb)
    return y.reshape(n, S, S, oc), s1, s2


def _head(y4, sc, sh, w5, n, *, B=128):
    B = min(B, n)
    w5rep = jnp.tile(w5.reshape(16, 512).astype(jnp.float32), (B, 1))
    out = pl.pallas_call(
        functools.partial(_head_kernel, B=B),
        out_shape=jax.ShapeDtypeStruct((n, 128), jnp.float32),
        grid=(n // B,),
        in_specs=[pl.BlockSpec((16 * B, 512), lambda i: (i, 0)),
                  pl.BlockSpec((1, 512), lambda i: (0, 0)),
                  pl.BlockSpec((1, 512), lambda i: (0, 0)),
                  pl.BlockSpec((16 * B, 512), lambda i: (0, 0))],
        out_specs=pl.BlockSpec((B, 128), lambda i: (i, 0)),
        compiler_params=pltpu.CompilerParams(
            dimension_semantics=("parallel",), vmem_limit_bytes=_VMEM_LIMIT),
    )(y4, sc, sh, w5rep)
    return out[:, :1].reshape(n, 1, 1, 1)


def kernel(w1, w2, w3, w4, w5, g2, g3, g4, b2, b3, b4, x):
    n = x.shape[0]
    x_nhwc = x.reshape(n, 64, 64, 1)

    y2, s1, s2 = _l1l2_conv(x_nhwc, w1, w2)
    sc2, sh2 = _fold_stats(s1, s2, g2, b2, n * 256)

    y3, s1, s2 = _fused_conv(y2, sc2, sh2, w3, S=8, B=16)
    sc3, sh3 = _fold_stats(s1, s2, g3, b3, n * 64)

    y4, s1, s2 = _fused_conv(y3, sc3, sh3, w4, S=4, B=32)
    sc4, sh4 = _fold_stats(s1, s2, g4, b4, n * 16)

    return _head(y4.reshape(n * 16, 512), sc4, sh4, w5, n)


# speedup vs baseline: 1.1216x; 1.1216x over previous
import functools

import jax
import jax.numpy as jnp
from jax.experimental import pallas as pl
from jax.experimental.pallas import tpu as pltpu

_EPS = 1e-5
_SLOPE = 0.2
_VMEM_LIMIT = 64 * 1024 * 1024


def _lrelu(v):
    return jnp.where(v >= 0.0, v, _SLOPE * v)


def _taps_stats(e, o, wa_ref, wb_ref, out_ref, s1_ref, s2_ref, *, S, B):
    c2 = e.shape[3]
    c = c2 // 2
    zrow = jnp.zeros((B, 1, S, c2), e.dtype)
    ep = jnp.concatenate([e, zrow], axis=1)
    op = jnp.concatenate([zrow, o], axis=1)
    zcol = jnp.zeros((B, S + 1, 1, c2), e.dtype)
    ep = jnp.concatenate([zcol, ep, zcol], axis=2)
    op = jnp.concatenate([zcol, op, zcol], axis=2)
    rows = (op, ep, op, ep)
    offs = (0, 0, 1, 1)
    acc = None
    for i in range(4):
        x = rows[i][:, offs[i] : offs[i] + S, :, :]
        a_sl = x[:, :, 1 : S + 1, :]
        d = jnp.dot(a_sl.reshape(B * S * S, c2), wa_ref[i],
                    preferred_element_type=jnp.float32)
        acc = d if acc is None else acc + d
        b_sl = jnp.concatenate(
            [x[:, :, 0:S, c:], x[:, :, 2 : S + 2, :c]], axis=3)
        acc = acc + jnp.dot(b_sl.reshape(B * S * S, c2), wb_ref[i],
                            preferred_element_type=jnp.float32)
    out_ref[...] = acc.astype(out_ref.dtype)
    s1 = jnp.sum(acc, axis=0, keepdims=True)
    s2 = jnp.sum(acc * acc, axis=0, keepdims=True)
    row = jax.lax.broadcasted_iota(jnp.int32, s1_ref.shape, 0)
    s1_ref[...] = jnp.where(row == 0, jnp.broadcast_to(s1, s1_ref.shape), 0.0)
    s2_ref[...] = jnp.where(row == 0, jnp.broadcast_to(s2, s2_ref.shape), 0.0)


def _l1l2_kernel(p_ref, w1_ref, wa_ref, wb_ref, out_ref, s1_ref, s2_ref,
                 *, S, B):
    pv = p_ref[...].reshape(B, 16, 2, 16, 32)
    acts = []
    for ph in (0, 1):
        pp = pv[:, :, ph].reshape(B * 16 * 16, 32)
        a = jnp.dot(pp, w1_ref[...], preferred_element_type=jnp.float32)
        acts.append(_lrelu(a).astype(jnp.bfloat16).reshape(B, 16, 16, 128))
    _taps_stats(acts[0], acts[1], wa_ref, wb_ref, out_ref, s1_ref, s2_ref,
                S=S, B=B)


def _fused_conv_kernel(y_ref, sc_ref, sh_ref, wa_ref, wb_ref,
                       out_ref, s1_ref, s2_ref, *, S, B, has_bn):
    c2 = y_ref.shape[3]
    yv = y_ref[...].reshape(B, S, 2, S, c2)
    e = yv[:, :, 0]
    o = yv[:, :, 1]
    if has_bn:
        e = _lrelu(e.astype(jnp.float32) * sc_ref[...] + sh_ref[...])
        o = _lrelu(o.astype(jnp.float32) * sc_ref[...] + sh_ref[...])
        e = e.astype(jnp.bfloat16)
        o = o.astype(jnp.bfloat16)
    _taps_stats(e, o, wa_ref, wb_ref, out_ref, s1_ref, s2_ref, S=S, B=B)


def _head_kernel(y_ref, sc_ref, sh_ref, w5_ref, o_ref, *, B):
    z = _lrelu(y_ref[...].astype(jnp.float32) * sc_ref[...] + sh_ref[...])
    zw = z * w5_ref[...]
    t = zw.reshape(B, 16, 512)
    s = jnp.sum(t, axis=2)
    logit = jnp.sum(s, axis=1, keepdims=True)
    prob = 1.0 / (1.0 + jnp.exp(-logit))
    o_ref[...] = jnp.broadcast_to(prob, o_ref.shape)


def _fold_stats(s1p, s2p, gamma, beta, m):
    s1 = jnp.sum(s1p, axis=0)
    s2 = jnp.sum(s2p, axis=0)
    mean = s1 / m
    var = jnp.maximum(s2 / m - mean * mean, 0.0)
    sc = gamma.astype(jnp.float32) * jax.lax.rsqrt(var + _EPS)
    sh = beta.astype(jnp.float32) - mean * sc
    return sc.reshape(1, -1), sh.reshape(1, -1)


def _l1l2_conv(x_nhwc, w1, w2, *, B=32):
    n = x_nhwc.shape[0]
    B = min(B, n)
    g = n // B
    oc = w2.shape[3]
    m = n * 256
    xp = jnp.pad(x_nhwc.astype(jnp.bfloat16), ((0, 0), (1, 1), (1, 1), (0, 0)))
    cols = [xp[:, i : i + 64 : 2, j : j + 64 : 2, 0] for i in range(4) for j in range(4)]
    patches = jnp.stack(cols, axis=-1)
    pv = patches.reshape(n, 32, 16, 32)

    wf = w1.reshape(16, 64)
    w1bd = jnp.zeros((32, 128), jnp.float32)
    w1bd = w1bd.at[0:16, 0:64].set(wf)
    w1bd = w1bd.at[16:32, 64:128].set(wf)
    w1bd = w1bd.astype(jnp.bfloat16)

    wr = w2.astype(jnp.bfloat16)
    wa = jnp.concatenate([wr[:, 1], wr[:, 2]], axis=1)
    wb = jnp.concatenate([wr[:, 0], wr[:, 3]], axis=1)

    y, s1, s2 = pl.pallas_call(
        functools.partial(_l1l2_kernel, S=16, B=B),
        out_shape=(jax.ShapeDtypeStruct((m, oc), jnp.bfloat16),
                   jax.ShapeDtypeStruct((8 * g, oc), jnp.float32),
                   jax.ShapeDtypeStruct((8 * g, oc), jnp.float32)),
        grid=(g,),
        in_specs=[pl.BlockSpec((B, 32, 16, 32), lambda i: (i, 0, 0, 0)),
                  pl.BlockSpec((32, 128), lambda i: (0, 0)),
                  pl.BlockSpec((4, 128, oc), lambda i: (0, 0, 0)),
                  pl.BlockSpec((4, 128, oc), lambda i: (0, 0, 0))],
        out_specs=(pl.BlockSpec((B * 256, oc), lambda i: (i, 0)),
                   pl.BlockSpec((8, oc), lambda i: (i, 0)),
                   pl.BlockSpec((8, oc), lambda i: (i, 0))),
        compiler_params=pltpu.CompilerParams(
            dimension_semantics=("parallel",), vmem_limit_bytes=_VMEM_LIMIT),
    )(pv, w1bd, wa, wb)
    return y.reshape(n, 16, 16, oc), s1, s2


def _fused_conv(y_prev, sc, sh, w, *, S, B, has_bn=True):
    n, _, _, c = y_prev.shape
    oc = w.shape[3]
    B = min(B, n)
    g = n // B
    m = n * S * S
    c2 = 2 * c
    yv = y_prev.reshape(n, 2 * S, S, c2)
    wr = w.astype(jnp.bfloat16)
    wa = jnp.concatenate([wr[:, 1], wr[:, 2]], axis=1)
    wb = jnp.concatenate([wr[:, 0], wr[:, 3]], axis=1)
    scd = jnp.tile(sc, (1, 2))
    shd = jnp.tile(sh, (1, 2))
    body = functools.partial(_fused_conv_kernel, S=S, B=B, has_bn=has_bn)
    y, s1, s2 = pl.pallas_call(
        body,
        out_shape=(jax.ShapeDtypeStruct((m, oc), jnp.bfloat16),
                   jax.ShapeDtypeStruct((8 * g, oc), jnp.float32),
                   jax.ShapeDtypeStruct((8 * g, oc), jnp.float32)),
        grid=(g,),
        in_specs=[pl.BlockSpec((B, 2 * S, S, c2), lambda i: (i, 0, 0, 0)),
                  pl.BlockSpec((1, c2), lambda i: (0, 0)),
                  pl.BlockSpec((1, c2), lambda i: (0, 0)),
                  pl.BlockSpec((4, c2, oc), lambda i: (0, 0, 0)),
                  pl.BlockSpec((4, c2, oc), lambda i: (0, 0, 0))],
        out_specs=(pl.BlockSpec((B * S * S, oc), lambda i: (i, 0)),
                   pl.BlockSpec((8, oc), lambda i: (i, 0)),
                   pl.BlockSpec((8, oc), lambda i: (i, 0))),
        compiler_params=pltpu.CompilerParams(
            dimension_semantics=("parallel",), vmem_limit_bytes=_VMEM_LIMIT),
    )(yv, scd, shd, wa, wb)
    return y.reshape(n, S, S, oc), s1, s2


def _head(y4, sc, sh, w5, n, *, B=128):
    B = min(B, n)
    w5rep = jnp.tile(w5.reshape(16, 512).astype(jnp.float32), (B, 1))
    out = pl.pallas_call(
        functools.partial(_head_kernel, B=B),
        out_shape=jax.ShapeDtypeStruct((n, 128), jnp.float32),
        grid=(n // B,),
        in_specs=[pl.BlockSpec((16 * B, 512), lambda i: (i, 0)),
                  pl.BlockSpec((1, 512), lambda i: (0, 0)),
                  pl.BlockSpec((1, 512), lambda i: (0, 0)),
                  pl.BlockSpec((16 * B, 512), lambda i: (0, 0))],
        out_specs=pl.BlockSpec((B, 128), lambda i: (i, 0)),
        compiler_params=pltpu.CompilerParams(
            dimension_semantics=("parallel",), vmem_limit_bytes=_VMEM_LIMIT),
    )(y4, sc, sh, w5rep)
    return out[:, :1].reshape(n, 1, 1, 1)


def kernel(w1, w2, w3, w4, w5, g2, g3, g4, b2, b3, b4, x):
    n = x.shape[0]
    x_nhwc = x.reshape(n, 64, 64, 1)

    y2, s1, s2 = _l1l2_conv(x_nhwc, w1, w2)
    sc2, sh2 = _fold_stats(s1, s2, g2, b2, n * 256)

    y3, s1, s2 = _fused_conv(y2, sc2, sh2, w3, S=8, B=32)
    sc3, sh3 = _fold_stats(s1, s2, g3, b3, n * 64)

    y4, s1, s2 = _fused_conv(y3, sc3, sh3, w4, S=4, B=64)
    sc4, sh4 = _fold_stats(s1, s2, g4, b4, n * 16)

    return _head(y4.reshape(n * 16, 512), sc4, sh4, w5, n)
